# baseline (speedup 1.0000x reference)
import numpy as np
import jax
import jax.numpy as jnp
from jax import lax
from jax.experimental import pallas as pl
from jax.experimental.pallas import tpu as pltpu
_INTERPRET = False

_SAMP = 8
_KNN = 3


def _fused_kernel(xs_ref, w_ref, b_ref, x_ref, o_ref, xsn_s, pt_s):
    @pl.when(pl.program_id(1) == 0)
    def _prep():
        xs = xs_ref[0]
        n2 = jnp.sqrt(jnp.sum(xs * xs, axis=0, keepdims=True))
        xsn_s[...] = xs / jnp.maximum(n2, 1e-12)
        parts = []
        for k in range(_KNN):
            parts.append(lax.dot_general(
                w_ref[k], xs,
                dimension_numbers=(((1,), (0,)), ((), ())),
                precision=lax.Precision.HIGHEST,
            ))
        pt = jnp.concatenate(parts, axis=1)
        pt_s[...] = pt + b_ref[:, :1] / np.float32(_KNN)

    xt = x_ref[0]
    nq = jnp.sqrt(jnp.sum(xt * xt, axis=0, keepdims=True))
    xq = xt / jnp.maximum(nq, 1e-12)
    sim = lax.dot_general(
        xsn_s[...], xq,
        dimension_numbers=(((0,), (0,)), ((), ())),
    )
    iota = lax.broadcasted_iota(jnp.int32, sim.shape, 0)
    sels = []
    for k in range(_KNN):
        v = jnp.max(sim, axis=0, keepdims=True)
        idx = jnp.where(sim == v, iota, 64)
        m = jnp.min(idx, axis=0, keepdims=True)
        sel = iota == m
        sels.append(sel.astype(jnp.bfloat16))
        if k < _KNN - 1:
            sim = jnp.where(sel, -jnp.inf, sim)
    oh = jnp.concatenate(sels, axis=0)
    out = lax.dot_general(
        pt_s[...], oh,
        dimension_numbers=(((1,), (0,)), ((), ())),
        preferred_element_type=jnp.float32,
    )
    o_ref[0] = jnp.maximum(out, 0.0)


def _copy4_kernel(a, b, c, d, oa, ob, oc, od):
    oa[...] = a[...]
    ob[...] = b[...]
    oc[...] = c[...]
    od[...] = d[...]


def kernel(x, W, b):
    B, C, H, Wd = x.shape
    N = H * Wd
    xi = np.round(np.linspace(0, H - 1, _SAMP)).astype(np.int32)
    yi = np.round(np.linspace(0, Wd - 1, _SAMP)).astype(np.int32)

    M = _SAMP * _SAMP
    TILE = 7168
    num_tiles = N // TILE

    def _run(x_l, W_l, b_l):
        B_l = x_l.shape[0]
        xr = jnp.concatenate(
            [lax.slice_in_dim(x_l, int(r), int(r) + 1, axis=2) for r in xi],
            axis=2,
        )
        xg = jnp.concatenate(
            [lax.slice_in_dim(xr, int(c), int(c) + 1, axis=3) for c in yi],
            axis=3,
        )
        xs = xg.reshape(B_l, C, M)
        Wr = jnp.transpose(W_l, (2, 0, 1))
        b2 = b_l.reshape(C, 1)
        xf = x_l.reshape(B_l, C, N)
        return pl.pallas_call(
            _fused_kernel,
            grid=(B_l, num_tiles),
            in_specs=[
                pl.BlockSpec((1, C, M), lambda i, j: (i, 0, 0)),
                pl.BlockSpec((_KNN, C, C), lambda i, j: (0, 0, 0)),
                pl.BlockSpec((C, 1), lambda i, j: (0, 0)),
                pl.BlockSpec((1, C, TILE), lambda i, j: (i, 0, j)),
            ],
            out_specs=pl.BlockSpec((1, C, TILE), lambda i, j: (i, 0, j)),
            out_shape=jax.ShapeDtypeStruct((B_l, C, N), jnp.float32),
            scratch_shapes=[
                pltpu.VMEM((C, M), jnp.float32),
                pltpu.VMEM((C, _KNN * M), jnp.float32),
            ],
            interpret=_INTERPRET,
        )(xs, Wr, b2, xf)

    if True:
        ROWS = 56
        x4 = x.reshape(4, 1176, 4096)
        xa, xb, xc, xd = x4[0], x4[1], x4[2], x4[3]
        outs = pl.pallas_call(
            _copy4_kernel,
            grid=(x4.shape[1] // ROWS,),
            in_specs=[pl.BlockSpec((ROWS, 4096), lambda i: (i, 0))] * 4,
            out_specs=[pl.BlockSpec((ROWS, 4096), lambda i: (i, 0))] * 4,
            out_shape=[jax.ShapeDtypeStruct(xa.shape, jnp.float32)] * 4,
            interpret=_INTERPRET,
        )(xa, xb, xc, xd)
        return jnp.stack(outs).reshape(B, C, H, Wd)
    out = _run(x, W, b)
    return out.reshape(B, C, H, Wd)


# speedup vs baseline: 1.4205x; 1.4205x over previous
import numpy as np
import jax
import jax.numpy as jnp
from jax import lax
from jax.experimental import pallas as pl
from jax.experimental.pallas import tpu as pltpu
_INTERPRET = False

_SAMP = 8
_KNN = 3


def _fused_kernel(xs_ref, w_ref, b_ref, x_ref, o_ref, xsn_s, pt_s):
    @pl.when(pl.program_id(1) == 0)
    def _prep():
        xs = xs_ref[0]
        n2 = jnp.sqrt(jnp.sum(xs * xs, axis=0, keepdims=True))
        xsn_s[...] = xs / jnp.maximum(n2, 1e-12)
        parts = []
        for k in range(_KNN):
            parts.append(lax.dot_general(
                w_ref[k], xs,
                dimension_numbers=(((1,), (0,)), ((), ())),
                precision=lax.Precision.HIGHEST,
            ))
        pt = jnp.concatenate(parts, axis=1)
        pt_s[...] = pt + b_ref[:, :1] / np.float32(_KNN)

    xt = x_ref[0]
    nq = jnp.sqrt(jnp.sum(xt * xt, axis=0, keepdims=True))
    xq = xt / jnp.maximum(nq, 1e-12)
    sim = lax.dot_general(
        xsn_s[...], xq,
        dimension_numbers=(((0,), (0,)), ((), ())),
    )
    iota = lax.broadcasted_iota(jnp.int32, sim.shape, 0)
    sels = []
    for k in range(_KNN):
        v = jnp.max(sim, axis=0, keepdims=True)
        idx = jnp.where(sim == v, iota, 64)
        m = jnp.min(idx, axis=0, keepdims=True)
        sel = iota == m
        sels.append(sel.astype(jnp.bfloat16))
        if k < _KNN - 1:
            sim = jnp.where(sel, -jnp.inf, sim)
    oh = jnp.concatenate(sels, axis=0)
    out = lax.dot_general(
        pt_s[...], oh,
        dimension_numbers=(((1,), (0,)), ((), ())),
        preferred_element_type=jnp.float32,
    )
    o_ref[0] = jnp.maximum(out, 0.0)


def kernel(x, W, b):
    B, C, H, Wd = x.shape
    N = H * Wd
    xi = np.round(np.linspace(0, H - 1, _SAMP)).astype(np.int32)
    yi = np.round(np.linspace(0, Wd - 1, _SAMP)).astype(np.int32)

    M = _SAMP * _SAMP
    TILE = 25088
    num_tiles = N // TILE

    def _run(x_l, W_l, b_l):
        B_l = x_l.shape[0]
        xr = jnp.concatenate(
            [lax.slice_in_dim(x_l, int(r), int(r) + 1, axis=2) for r in xi],
            axis=2,
        )
        xg = jnp.concatenate(
            [lax.slice_in_dim(xr, int(c), int(c) + 1, axis=3) for c in yi],
            axis=3,
        )
        xs = xg.reshape(B_l, C, M)
        Wr = jnp.transpose(W_l, (2, 0, 1))
        b2 = b_l.reshape(C, 1)
        xf = x_l.reshape(B_l, C, N)
        return pl.pallas_call(
            _fused_kernel,
            grid=(B_l, num_tiles),
            in_specs=[
                pl.BlockSpec((1, C, M), lambda i, j: (i, 0, 0)),
                pl.BlockSpec((_KNN, C, C), lambda i, j: (0, 0, 0)),
                pl.BlockSpec((C, 1), lambda i, j: (0, 0)),
                pl.BlockSpec((1, C, TILE), lambda i, j: (i, 0, j)),
            ],
            out_specs=pl.BlockSpec((1, C, TILE), lambda i, j: (i, 0, j)),
            out_shape=jax.ShapeDtypeStruct((B_l, C, N), jnp.float32),
            scratch_shapes=[
                pltpu.VMEM((C, M), jnp.float32),
                pltpu.VMEM((C, _KNN * M), jnp.float32),
            ],
            interpret=_INTERPRET,
        )(xs, Wr, b2, xf)

    out = _run(x, W, b)
    return out.reshape(B, C, H, Wd)


# speedup vs baseline: 1.4256x; 1.0036x over previous
import numpy as np
import jax
import jax.numpy as jnp
from jax import lax
from jax.experimental import pallas as pl
from jax.experimental.pallas import tpu as pltpu
_INTERPRET = False

_SAMP = 8
_KNN = 3


def _fused_kernel(xs_ref, w_ref, b_ref, x_ref, o_ref, xsn_s, pt_s):
    @pl.when(pl.program_id(1) == 0)
    def _prep():
        xs = xs_ref[0]
        n2 = jnp.sqrt(jnp.sum(xs * xs, axis=0, keepdims=True))
        xsn_s[...] = xs / jnp.maximum(n2, 1e-12)
        parts = []
        for k in range(_KNN):
            parts.append(lax.dot_general(
                w_ref[k], xs,
                dimension_numbers=(((1,), (0,)), ((), ())),
                precision=lax.Precision.HIGHEST,
            ))
        pt = jnp.concatenate(parts, axis=1)
        pt_s[...] = pt + b_ref[:, :1] / np.float32(_KNN)

    xt = x_ref[0]
    nq = jnp.sqrt(jnp.sum(xt * xt, axis=0, keepdims=True))
    xq = xt / jnp.maximum(nq, 1e-12)
    sim = lax.dot_general(
        xsn_s[...], xq,
        dimension_numbers=(((0,), (0,)), ((), ())),
    )
    iota = lax.broadcasted_iota(jnp.int32, sim.shape, 0)
    sels = []
    for k in range(_KNN):
        v = jnp.max(sim, axis=0, keepdims=True)
        idx = jnp.where(sim == v, iota, 64)
        m = jnp.min(idx, axis=0, keepdims=True)
        sel = iota == m
        sels.append(sel.astype(jnp.bfloat16))
        if k < _KNN - 1:
            sim = jnp.where(sel, -jnp.inf, sim)
    oh = jnp.concatenate(sels, axis=0)
    out = lax.dot_general(
        pt_s[...], oh,
        dimension_numbers=(((1,), (0,)), ((), ())),
        preferred_element_type=jnp.float32,
    )
    o_ref[0] = jnp.maximum(out, 0.0)


def kernel(x, W, b):
    B, C, H, Wd = x.shape
    N = H * Wd
    xi = np.round(np.linspace(0, H - 1, _SAMP)).astype(np.int32)
    yi = np.round(np.linspace(0, Wd - 1, _SAMP)).astype(np.int32)

    M = _SAMP * _SAMP
    TILE = 12544
    num_tiles = N // TILE

    def _run(x_l, W_l, b_l):
        B_l = x_l.shape[0]
        xr = jnp.concatenate(
            [lax.slice_in_dim(x_l, int(r), int(r) + 1, axis=2) for r in xi],
            axis=2,
        )
        xg = jnp.concatenate(
            [lax.slice_in_dim(xr, int(c), int(c) + 1, axis=3) for c in yi],
            axis=3,
        )
        xs = xg.reshape(B_l, C, M)
        Wr = jnp.transpose(W_l, (2, 0, 1))
        b2 = b_l.reshape(C, 1)
        xf = x_l.reshape(B_l, C, N)
        return pl.pallas_call(
            _fused_kernel,
            grid=(B_l, num_tiles),
            in_specs=[
                pl.BlockSpec((1, C, M), lambda i, j: (i, 0, 0)),
                pl.BlockSpec((_KNN, C, C), lambda i, j: (0, 0, 0)),
                pl.BlockSpec((C, 1), lambda i, j: (0, 0)),
                pl.BlockSpec((1, C, TILE), lambda i, j: (i, 0, j)),
            ],
            out_specs=pl.BlockSpec((1, C, TILE), lambda i, j: (i, 0, j)),
            out_shape=jax.ShapeDtypeStruct((B_l, C, N), jnp.float32),
            scratch_shapes=[
                pltpu.VMEM((C, M), jnp.float32),
                pltpu.VMEM((C, _KNN * M), jnp.float32),
            ],
            interpret=_INTERPRET,
        )(xs, Wr, b2, xf)

    out = _run(x, W, b)
    return out.reshape(B, C, H, Wd)
